# baseline (speedup 1.0000x reference)
import functools

import jax
import jax.numpy as jnp
from jax import lax
from jax.experimental import pallas as pl
from jax.experimental.pallas import tpu as pltpu
from jax.experimental.pallas import tpu_sc as plsc

B = 16384
D = 64
NC = 2
NS = 16
NW = NC * NS
BPW = B // NW
CHUNK = 128
NCHUNK = BPW // CHUNK


def _l2_normalize_rows(rows_ref):
    def body(r, carry):
        vs = []
        sq = jnp.zeros((16,), jnp.float32)
        for j in range(D // 16):
            v = rows_ref[r, pl.ds(j * 16, 16)]
            vs.append(v)
            sq = sq + v * v
        s = jnp.sum(sq)
        i = lax.bitcast_convert_type(s, jnp.int32)
        i = jnp.int32(0x5F3759DF) - lax.shift_right_logical(i, 1)
        y = lax.bitcast_convert_type(i, jnp.float32)
        for _ in range(3):
            y = y * (jnp.float32(1.5) - jnp.float32(0.5) * s * y * y)
        norm = s * y
        inv = jnp.where(norm > jnp.float32(1e-12), y, jnp.float32(1e12))
        for j in range(D // 16):
            rows_ref[r, pl.ds(j * 16, 16)] = vs[j] * inv
        return carry

    lax.fori_loop(0, BPW, body, jnp.int32(0))


_mesh = plsc.VectorSubcoreMesh(core_axis_name="c", subcore_axis_name="s")


@functools.partial(
    pl.kernel,
    mesh=_mesh,
    out_type=[
        jax.ShapeDtypeStruct((B, D), jnp.float32),
        jax.ShapeDtypeStruct((B, D), jnp.float32),
    ],
    scratch_types=[
        pltpu.VMEM((BPW,), jnp.int32),
        pltpu.VMEM((BPW,), jnp.int32),
        pltpu.VMEM((BPW, D), jnp.float32),
        pltpu.VMEM((BPW, D), jnp.float32),
        pltpu.SemaphoreType.DMA,
        pltpu.SemaphoreType.DMA,
    ],
    compiler_params=pltpu.CompilerParams(
        needs_layout_passes=False, use_tc_tiling_on_sc=False),
)
def _embed_kernel(uids_hbm, iids_hbm, utab_hbm, itab_hbm, uout_hbm, iout_hbm,
                  uidx_v, iidx_v, urows_v, irows_v, sem_u, sem_i):
    wid = lax.axis_index("s") * NC + lax.axis_index("c")
    base = wid * BPW

    pltpu.sync_copy(uids_hbm.at[pl.ds(base, BPW)], uidx_v)
    pltpu.sync_copy(iids_hbm.at[pl.ds(base, BPW)], iidx_v)

    u_caps = []
    i_caps = []
    for k in range(NCHUNK):
        u_caps.append(pltpu.async_copy(
            utab_hbm.at[uidx_v.at[pl.ds(k * CHUNK, CHUNK)]],
            urows_v.at[pl.ds(k * CHUNK, CHUNK)], sem_u))
    for k in range(NCHUNK):
        i_caps.append(pltpu.async_copy(
            itab_hbm.at[iidx_v.at[pl.ds(k * CHUNK, CHUNK)]],
            irows_v.at[pl.ds(k * CHUNK, CHUNK)], sem_i))

    for c in u_caps:
        c.wait()
    _l2_normalize_rows(urows_v)
    u_out = pltpu.async_copy(urows_v, uout_hbm.at[pl.ds(base, BPW)], sem_u)

    for c in i_caps:
        c.wait()
    _l2_normalize_rows(irows_v)
    pltpu.sync_copy(irows_v, iout_hbm.at[pl.ds(base, BPW)])
    u_out.wait()


def kernel(user_ids, item_ids, user_table, item_table):
    u, it = _embed_kernel(user_ids.astype(jnp.int32), item_ids.astype(jnp.int32),
                          user_table, item_table)
    return (u, it)


# speedup vs baseline: 1.5564x; 1.5564x over previous
import functools

import jax
import jax.numpy as jnp
from jax import lax
from jax.experimental import pallas as pl
from jax.experimental.pallas import tpu as pltpu
from jax.experimental.pallas import tpu_sc as plsc

B = 16384
D = 64
NC = 2
NS = 16
NW = NC * NS
BPW = B // NW
NPASS = 2
CROWS = BPW // NPASS
NGROUP = CROWS // 16


def _fire_row_gathers(tab_hbm, idx_v, buf, sem, chunk_base):
    def group(g, carry):
        idxv = idx_v[pl.ds(chunk_base + g * 16, 16)]
        for j in range(16):
            r = idxv[j]
            pltpu.async_copy(
                tab_hbm.at[pl.ds(r, 1), :],
                buf.at[pl.ds(g * 16 + j, 1), :],
                sem,
            )
        return carry

    lax.fori_loop(0, NGROUP, group, jnp.int32(0))


def _drain(tab_hbm, buf, sem):
    pltpu.make_async_copy(tab_hbm.at[pl.ds(0, CROWS), :], buf, sem).wait()


def _l2_normalize_rows(rows_ref):
    def body(r, carry):
        vs = []
        sq = jnp.zeros((16,), jnp.float32)
        for j in range(D // 16):
            v = rows_ref[r, pl.ds(j * 16, 16)]
            vs.append(v)
            sq = sq + v * v
        s = jnp.sum(sq)
        i = lax.bitcast_convert_type(s, jnp.int32)
        i = jnp.int32(0x5F3759DF) - lax.shift_right_logical(i, 1)
        y = lax.bitcast_convert_type(i, jnp.float32)
        for _ in range(3):
            y = y * (jnp.float32(1.5) - jnp.float32(0.5) * s * y * y)
        norm = s * y
        inv = jnp.where(norm > jnp.float32(1e-12), y, jnp.float32(1e12))
        for j in range(D // 16):
            rows_ref[r, pl.ds(j * 16, 16)] = vs[j] * inv
        return carry

    lax.fori_loop(0, CROWS, body, jnp.int32(0))


_mesh = plsc.VectorSubcoreMesh(core_axis_name="c", subcore_axis_name="s")


@functools.partial(
    pl.kernel,
    mesh=_mesh,
    out_type=[
        jax.ShapeDtypeStruct((B, D), jnp.float32),
        jax.ShapeDtypeStruct((B, D), jnp.float32),
    ],
    scratch_types=[
        pltpu.VMEM((BPW,), jnp.int32),
        pltpu.VMEM((BPW,), jnp.int32),
        pltpu.VMEM((CROWS, D), jnp.float32),
        pltpu.VMEM((CROWS, D), jnp.float32),
        pltpu.SemaphoreType.DMA,
        pltpu.SemaphoreType.DMA,
        pltpu.SemaphoreType.DMA,
        pltpu.SemaphoreType.DMA,
    ],
    compiler_params=pltpu.CompilerParams(needs_layout_passes=False),
)
def _embed_kernel(uids_hbm, iids_hbm, utab_hbm, itab_hbm, uout_hbm, iout_hbm,
                  uidx_v, iidx_v, ubuf, ibuf, sem_gu, sem_gi, sem_su, sem_si):
    wid = lax.axis_index("s") * NC + lax.axis_index("c")
    base = wid * BPW

    pltpu.sync_copy(uids_hbm.at[pl.ds(base, BPW)], uidx_v)
    pltpu.sync_copy(iids_hbm.at[pl.ds(base, BPW)], iidx_v)

    u_store = i_store = None
    for p in range(NPASS):
        if p:
            u_store.wait()
            i_store.wait()
        _fire_row_gathers(utab_hbm, uidx_v, ubuf, sem_gu, p * CROWS)
        _fire_row_gathers(itab_hbm, iidx_v, ibuf, sem_gi, p * CROWS)
        _drain(utab_hbm, ubuf, sem_gu)
        _l2_normalize_rows(ubuf)
        u_store = pltpu.async_copy(
            ubuf, uout_hbm.at[pl.ds(base + p * CROWS, CROWS)], sem_su)
        _drain(itab_hbm, ibuf, sem_gi)
        _l2_normalize_rows(ibuf)
        i_store = pltpu.async_copy(
            ibuf, iout_hbm.at[pl.ds(base + p * CROWS, CROWS)], sem_si)
    u_store.wait()
    i_store.wait()


def kernel(user_ids, item_ids, user_table, item_table):
    u, it = _embed_kernel(user_ids.astype(jnp.int32), item_ids.astype(jnp.int32),
                          user_table, item_table)
    return (u, it)
